# baseline (speedup 1.0000x reference)
import functools

import jax
import jax.numpy as jnp
from jax import lax
from jax.experimental import pallas as pl
from jax.experimental.pallas import tpu as pltpu
from jax.experimental.pallas import tpu_sc as plsc

_BATCH = 16384
_D = 64
_NC = 2
_NS = 16
_NW = _NC * _NS
_BPW = _BATCH // _NW
_CHUNK = 128
_NCHUNK = _BPW // _CHUNK


def _sc_gather4(uid_hbm, iid_hbm, mu_hbm, mi_hbm, gu_hbm, gi_hbm,
                out_mu, out_mi, out_gu, out_gi,
                idx_u, idx_i, buf0, buf1, sem0, sem1):
    wid = lax.axis_index("s") * _NC + lax.axis_index("c")
    base = wid * _BPW
    pltpu.sync_copy(uid_hbm.at[pl.ds(base, _BPW)], idx_u)
    pltpu.sync_copy(iid_hbm.at[pl.ds(base, _BPW)], idx_i)

    tasks = []
    for c in range(_NCHUNK):
        off = c * _CHUNK
        tasks.append((gu_hbm, idx_u, out_gu, off))
        tasks.append((gi_hbm, idx_i, out_gi, off))
        tasks.append((mu_hbm, idx_u, out_mu, off))
        tasks.append((mi_hbm, idx_i, out_mi, off))

    bufs = (buf0, buf1)
    sems = (sem0, sem1)
    pending = [None, None]
    for t, (table, idx, out, off) in enumerate(tasks):
        slot = t % 2
        if pending[slot] is not None:
            cp, pout, poff = pending[slot]
            cp.wait()
            pltpu.sync_copy(bufs[slot], pout.at[pl.ds(base + poff, _CHUNK)])
        cp = pltpu.async_copy(table.at[idx.at[pl.ds(off, _CHUNK)]],
                              bufs[slot], sems[slot])
        pending[slot] = (cp, out, off)
    for slot in range(2):
        cp, pout, poff = pending[slot]
        cp.wait()
        pltpu.sync_copy(bufs[slot], pout.at[pl.ds(base + poff, _CHUNK)])


@functools.cache
def _gather4_built():
    return pl.kernel(
        _sc_gather4,
        mesh=plsc.VectorSubcoreMesh(core_axis_name="c", subcore_axis_name="s"),
        out_type=[jax.ShapeDtypeStruct((_BATCH, _D), jnp.float32)] * 4,
        scratch_types=[
            pltpu.VMEM((_BPW,), jnp.int32),
            pltpu.VMEM((_BPW,), jnp.int32),
            pltpu.VMEM((_CHUNK, _D), jnp.float32),
            pltpu.VMEM((_CHUNK, _D), jnp.float32),
            pltpu.SemaphoreType.DMA,
            pltpu.SemaphoreType.DMA,
        ],
        compiler_params=pltpu.CompilerParams(use_tc_tiling_on_sc=False),
    )


_BLK = 2048


def _mlp_body(mu_ref, mi_ref, gu_ref, gi_ref,
              w1a_ref, w1b_ref, b1_ref, w2_ref, b2_ref, w3_ref, b3_ref,
              wog_ref, wom_ref, bo_ref, out_ref):
    f32 = jnp.float32
    h1 = jnp.dot(mu_ref[...], w1a_ref[...], preferred_element_type=f32)
    h1 = h1 + jnp.dot(mi_ref[...], w1b_ref[...], preferred_element_type=f32)
    h1 = jnp.maximum(h1 + b1_ref[...], 0.0)
    h2 = jnp.maximum(
        jnp.dot(h1, w2_ref[...], preferred_element_type=f32) + b2_ref[...], 0.0)
    h3 = jnp.maximum(
        jnp.dot(h2, w3_ref[...], preferred_element_type=f32) + b3_ref[...], 0.0)
    gmf = gu_ref[...] * gi_ref[...]
    logits = (jnp.dot(gmf, wog_ref[...], preferred_element_type=f32)
              + jnp.dot(h3, wom_ref[...], preferred_element_type=f32)
              + bo_ref[...])
    out_ref[...] = logits[:, 0]


def _mlp(mu, mi, gu, gi, w1a, w1b, b1, w2, b2, w3, b3, wog, wom, bo):
    n_blocks = _BATCH // _BLK
    emb_spec = pl.BlockSpec((_BLK, _D), lambda i: (i, 0))

    def full(a):
        return pl.BlockSpec(a.shape, lambda i: (0,) * a.ndim)

    return pl.pallas_call(
        _mlp_body,
        grid=(n_blocks,),
        in_specs=[emb_spec, emb_spec, emb_spec, emb_spec,
                  full(w1a), full(w1b), full(b1), full(w2), full(b2),
                  full(w3), full(b3), full(wog), full(wom), full(bo)],
        out_specs=pl.BlockSpec((_BLK,), lambda i: (i,)),
        out_shape=jax.ShapeDtypeStruct((_BATCH,), jnp.float32),
    )(mu, mi, gu, gi, w1a, w1b, b1, w2, b2, w3, b3, wog, wom, bo)


def kernel(x, mlp_user_emb, mlp_item_emb, gmf_user_emb, gmf_item_emb,
           W1, b1, W2, b2, W3, b3, W_out, b_out):
    uid = x[:, 0]
    iid = x[:, 1]
    mu, mi, gu, gi = _gather4_built()(uid, iid, mlp_user_emb, mlp_item_emb,
                                      gmf_user_emb, gmf_item_emb)
    return _mlp(mu, mi, gu, gi,
                W1[:_D], W1[_D:], b1.reshape(1, -1),
                W2, b2.reshape(1, -1), W3, b3.reshape(1, -1),
                W_out[:_D], W_out[_D:], b_out.reshape(1, 1))


# speedup vs baseline: 1.5103x; 1.5103x over previous
import functools

import jax
import jax.numpy as jnp
from jax import lax
from jax.experimental import pallas as pl
from jax.experimental.pallas import tpu as pltpu
from jax.experimental.pallas import tpu_sc as plsc

_BATCH = 16384
_D = 64
_NC = 2
_NS = 16
_NW = _NC * _NS
_BPW = _BATCH // _NW
_G = 16


def _sc_gather4(uid_hbm, iid_hbm, mu_hbm, mi_hbm, gu_hbm, gi_hbm,
                out_mu, out_mi, out_gu, out_gi,
                idx_u, idx_i, rows, sem):
    wid = lax.axis_index("s") * _NC + lax.axis_index("c")
    base = wid * _BPW
    pltpu.sync_copy(uid_hbm.at[pl.ds(base, _BPW)], idx_u)
    pltpu.sync_copy(iid_hbm.at[pl.ds(base, _BPW)], idx_i)

    def gather_one(table, idx, out):
        def body(g, _):
            v = idx[pl.ds(g * _G, _G)]
            for k in range(_G):
                pltpu.async_copy(table.at[pl.ds(v[k], 1)],
                                 rows.at[pl.ds(g * _G + k, 1)], sem)
            return 0

        lax.fori_loop(0, _BPW // _G, body, 0, unroll=False)
        pltpu.make_async_copy(table.at[pl.ds(0, _BPW)], rows, sem).wait()
        pltpu.sync_copy(rows, out.at[pl.ds(base, _BPW)])

    gather_one(gu_hbm, idx_u, out_gu)
    gather_one(gi_hbm, idx_i, out_gi)
    gather_one(mu_hbm, idx_u, out_mu)
    gather_one(mi_hbm, idx_i, out_mi)


@functools.cache
def _gather4_built():
    return pl.kernel(
        _sc_gather4,
        mesh=plsc.VectorSubcoreMesh(core_axis_name="c", subcore_axis_name="s"),
        out_type=[jax.ShapeDtypeStruct((_BATCH, _D), jnp.float32)] * 4,
        scratch_types=[
            pltpu.VMEM((_BPW,), jnp.int32),
            pltpu.VMEM((_BPW,), jnp.int32),
            pltpu.VMEM((_BPW, _D), jnp.float32),
            pltpu.SemaphoreType.DMA,
        ],
    )


_BLK = 2048


def _mlp_body(mu_ref, mi_ref, gu_ref, gi_ref,
              w1a_ref, w1b_ref, b1_ref, w2_ref, b2_ref, w3_ref, b3_ref,
              wog_ref, wom_ref, bo_ref, out_ref):
    f32 = jnp.float32
    h1 = jnp.dot(mu_ref[...], w1a_ref[...], preferred_element_type=f32)
    h1 = h1 + jnp.dot(mi_ref[...], w1b_ref[...], preferred_element_type=f32)
    h1 = jnp.maximum(h1 + b1_ref[...], 0.0)
    h2 = jnp.maximum(
        jnp.dot(h1, w2_ref[...], preferred_element_type=f32) + b2_ref[...], 0.0)
    h3 = jnp.maximum(
        jnp.dot(h2, w3_ref[...], preferred_element_type=f32) + b3_ref[...], 0.0)
    gmf = gu_ref[...] * gi_ref[...]
    logits = (jnp.dot(gmf, wog_ref[...], preferred_element_type=f32)
              + jnp.dot(h3, wom_ref[...], preferred_element_type=f32)
              + bo_ref[...])
    out_ref[...] = logits[:, 0]


def _mlp(mu, mi, gu, gi, w1a, w1b, b1, w2, b2, w3, b3, wog, wom, bo):
    n_blocks = _BATCH // _BLK
    emb_spec = pl.BlockSpec((_BLK, _D), lambda i: (i, 0))

    def full(a):
        return pl.BlockSpec(a.shape, lambda i: (0,) * a.ndim)

    return pl.pallas_call(
        _mlp_body,
        grid=(n_blocks,),
        in_specs=[emb_spec, emb_spec, emb_spec, emb_spec,
                  full(w1a), full(w1b), full(b1), full(w2), full(b2),
                  full(w3), full(b3), full(wog), full(wom), full(bo)],
        out_specs=pl.BlockSpec((_BLK,), lambda i: (i,)),
        out_shape=jax.ShapeDtypeStruct((_BATCH,), jnp.float32),
    )(mu, mi, gu, gi, w1a, w1b, b1, w2, b2, w3, b3, wog, wom, bo)


def kernel(x, mlp_user_emb, mlp_item_emb, gmf_user_emb, gmf_item_emb,
           W1, b1, W2, b2, W3, b3, W_out, b_out):
    uid = x[:, 0]
    iid = x[:, 1]
    mu, mi, gu, gi = _gather4_built()(uid, iid, mlp_user_emb, mlp_item_emb,
                                      gmf_user_emb, gmf_item_emb)
    return _mlp(mu, mi, gu, gi,
                W1[:_D], W1[_D:], b1.reshape(1, -1),
                W2, b2.reshape(1, -1), W3, b3.reshape(1, -1),
                W_out[:_D], W_out[_D:], b_out.reshape(1, 1))


# speedup vs baseline: 1.8645x; 1.2345x over previous
import functools

import jax
import jax.numpy as jnp
from jax import lax
from jax.experimental import pallas as pl
from jax.experimental.pallas import tpu as pltpu
from jax.experimental.pallas import tpu_sc as plsc

_BATCH = 16384
_D = 64
_N = 1000000
_NC = 2
_NS = 16
_NW = _NC * _NS
_BPW = _BATCH // _NW
_G = 16

_RBLK = 8192


def _repack_body(tt_ref, eye_ref, out_ref):
    out_ref[...] = jax.lax.dot_general(
        tt_ref[...], eye_ref[...], (((0,), (0,)), ((), ())),
        preferred_element_type=jnp.float32)


@functools.cache
def _repack_built():
    grid = (_N + _RBLK - 1) // _RBLK
    return pl.pallas_call(
        _repack_body,
        grid=(grid,),
        in_specs=[pl.BlockSpec((_D, _RBLK), lambda i: (0, i)),
                  pl.BlockSpec((_D, _D), lambda i: (0, 0))],
        out_specs=pl.BlockSpec((_RBLK, _D), lambda i: (i, 0)),
        out_shape=jax.ShapeDtypeStruct((_N, _D), jnp.float32),
    )


def _sc_gather4(uid_hbm, iid_hbm, mu_hbm, mi_hbm, gu_hbm, gi_hbm,
                out_mu, out_mi, out_gu, out_gi,
                idx_u, idx_i, rows, sem):
    wid = lax.axis_index("s") * _NC + lax.axis_index("c")
    base = wid * _BPW
    pltpu.sync_copy(uid_hbm.at[pl.ds(base, _BPW)], idx_u)
    pltpu.sync_copy(iid_hbm.at[pl.ds(base, _BPW)], idx_i)

    def gather_one(table, idx, out):
        def body(g, _):
            v = idx[pl.ds(g * _G, _G)]
            for k in range(_G):
                pltpu.async_copy(table.at[pl.ds(v[k], 1)],
                                 rows.at[pl.ds(g * _G + k, 1)], sem)
            return 0

        lax.fori_loop(0, _BPW // _G, body, 0, unroll=False)
        pltpu.make_async_copy(table.at[pl.ds(0, _BPW)], rows, sem).wait()
        pltpu.sync_copy(rows, out.at[pl.ds(base, _BPW)])

    gather_one(gu_hbm, idx_u, out_gu)
    gather_one(gi_hbm, idx_i, out_gi)
    gather_one(mu_hbm, idx_u, out_mu)
    gather_one(mi_hbm, idx_i, out_mi)


@functools.cache
def _gather4_built():
    return pl.kernel(
        _sc_gather4,
        mesh=plsc.VectorSubcoreMesh(core_axis_name="c", subcore_axis_name="s"),
        out_type=[jax.ShapeDtypeStruct((_BATCH, _D), jnp.float32)] * 4,
        scratch_types=[
            pltpu.VMEM((_BPW,), jnp.int32),
            pltpu.VMEM((_BPW,), jnp.int32),
            pltpu.VMEM((_BPW, _D), jnp.float32),
            pltpu.SemaphoreType.DMA,
        ],
    )


_BLK = 2048


def _mlp_body(mu_ref, mi_ref, gu_ref, gi_ref,
              w1a_ref, w1b_ref, b1_ref, w2_ref, b2_ref, w3_ref, b3_ref,
              wog_ref, wom_ref, bo_ref, out_ref):
    f32 = jnp.float32
    h1 = jnp.dot(mu_ref[...], w1a_ref[...], preferred_element_type=f32)
    h1 = h1 + jnp.dot(mi_ref[...], w1b_ref[...], preferred_element_type=f32)
    h1 = jnp.maximum(h1 + b1_ref[...], 0.0)
    h2 = jnp.maximum(
        jnp.dot(h1, w2_ref[...], preferred_element_type=f32) + b2_ref[...], 0.0)
    h3 = jnp.maximum(
        jnp.dot(h2, w3_ref[...], preferred_element_type=f32) + b3_ref[...], 0.0)
    gmf = gu_ref[...] * gi_ref[...]
    logits = (jnp.dot(gmf, wog_ref[...], preferred_element_type=f32)
              + jnp.dot(h3, wom_ref[...], preferred_element_type=f32)
              + bo_ref[...])
    out_ref[...] = logits[:, 0]


def _mlp(mu, mi, gu, gi, w1a, w1b, b1, w2, b2, w3, b3, wog, wom, bo):
    n_blocks = _BATCH // _BLK
    emb_spec = pl.BlockSpec((_BLK, _D), lambda i: (i, 0))

    def full(a):
        return pl.BlockSpec(a.shape, lambda i: (0,) * a.ndim)

    return pl.pallas_call(
        _mlp_body,
        grid=(n_blocks,),
        in_specs=[emb_spec, emb_spec, emb_spec, emb_spec,
                  full(w1a), full(w1b), full(b1), full(w2), full(b2),
                  full(w3), full(b3), full(wog), full(wom), full(bo)],
        out_specs=pl.BlockSpec((_BLK,), lambda i: (i,)),
        out_shape=jax.ShapeDtypeStruct((_BATCH,), jnp.float32),
    )(mu, mi, gu, gi, w1a, w1b, b1, w2, b2, w3, b3, wog, wom, bo)


def kernel(x, mlp_user_emb, mlp_item_emb, gmf_user_emb, gmf_item_emb,
           W1, b1, W2, b2, W3, b3, W_out, b_out):
    uid = x[:, 0]
    iid = x[:, 1]
    eye = jnp.eye(_D, dtype=jnp.float32)
    repack = lambda t: _repack_built()(t.T, eye)
    mu, mi, gu, gi = _gather4_built()(
        uid, iid, repack(mlp_user_emb), repack(mlp_item_emb),
        repack(gmf_user_emb), repack(gmf_item_emb))
    return _mlp(mu, mi, gu, gi,
                W1[:_D], W1[_D:], b1.reshape(1, -1),
                W2, b2.reshape(1, -1), W3, b3.reshape(1, -1),
                W_out[:_D], W_out[_D:], b_out.reshape(1, 1))


# speedup vs baseline: 2.1126x; 1.1331x over previous
import functools

import jax
import jax.numpy as jnp
from jax import lax
from jax.experimental import pallas as pl
from jax.experimental.pallas import tpu as pltpu
from jax.experimental.pallas import tpu_sc as plsc

_BATCH = 16384
_D = 64
_N = 1000000
_NC = 2
_NS = 16
_NW = _NC * _NS
_BPW = _BATCH // _NW
_G = 16

_RBLK = 32768


def _repack_body(tt_ref, eye_ref, out_ref):
    out_ref[...] = jax.lax.dot_general(
        tt_ref[...], eye_ref[...], (((0,), (0,)), ((), ())),
        preferred_element_type=jnp.float32)


@functools.cache
def _repack_built():
    grid = (_N + _RBLK - 1) // _RBLK
    return pl.pallas_call(
        _repack_body,
        grid=(grid,),
        in_specs=[pl.BlockSpec((_D, _RBLK), lambda i: (0, i)),
                  pl.BlockSpec((_D, _D), lambda i: (0, 0))],
        out_specs=pl.BlockSpec((_RBLK, _D), lambda i: (i, 0)),
        out_shape=jax.ShapeDtypeStruct((_N, _D), jnp.float32),
    )


def _sc_gather4(uid_hbm, iid_hbm, mu_hbm, mi_hbm, gu_hbm, gi_hbm,
                out_mu, out_mi, out_gu, out_gi,
                idx_u, idx_i, rows, sem):
    wid = lax.axis_index("s") * _NC + lax.axis_index("c")
    base = wid * _BPW
    pltpu.sync_copy(uid_hbm.at[pl.ds(base, _BPW)], idx_u)
    pltpu.sync_copy(iid_hbm.at[pl.ds(base, _BPW)], idx_i)

    def gather_one(table, idx, out):
        def body(g, _):
            v = idx[pl.ds(g * _G, _G)]
            for k in range(_G):
                pltpu.async_copy(table.at[pl.ds(v[k], 1)],
                                 rows.at[pl.ds(g * _G + k, 1)], sem)
            return 0

        lax.fori_loop(0, _BPW // _G, body, 0, unroll=False)
        pltpu.make_async_copy(table.at[pl.ds(0, _BPW)], rows, sem).wait()
        pltpu.sync_copy(rows, out.at[pl.ds(base, _BPW)])

    gather_one(gu_hbm, idx_u, out_gu)
    gather_one(gi_hbm, idx_i, out_gi)
    gather_one(mu_hbm, idx_u, out_mu)
    gather_one(mi_hbm, idx_i, out_mi)


@functools.cache
def _gather4_built():
    return pl.kernel(
        _sc_gather4,
        mesh=plsc.VectorSubcoreMesh(core_axis_name="c", subcore_axis_name="s"),
        out_type=[jax.ShapeDtypeStruct((_BATCH, _D), jnp.float32)] * 4,
        scratch_types=[
            pltpu.VMEM((_BPW,), jnp.int32),
            pltpu.VMEM((_BPW,), jnp.int32),
            pltpu.VMEM((_BPW, _D), jnp.float32),
            pltpu.SemaphoreType.DMA,
        ],
    )


_BLK = 2048


def _mlp_body(mu_ref, mi_ref, gu_ref, gi_ref,
              w1a_ref, w1b_ref, b1_ref, w2_ref, b2_ref, w3_ref, b3_ref,
              wog_ref, wom_ref, bo_ref, out_ref):
    f32 = jnp.float32
    h1 = jnp.dot(mu_ref[...], w1a_ref[...], preferred_element_type=f32)
    h1 = h1 + jnp.dot(mi_ref[...], w1b_ref[...], preferred_element_type=f32)
    h1 = jnp.maximum(h1 + b1_ref[...], 0.0)
    h2 = jnp.maximum(
        jnp.dot(h1, w2_ref[...], preferred_element_type=f32) + b2_ref[...], 0.0)
    h3 = jnp.maximum(
        jnp.dot(h2, w3_ref[...], preferred_element_type=f32) + b3_ref[...], 0.0)
    gmf = gu_ref[...] * gi_ref[...]
    logits = (jnp.dot(gmf, wog_ref[...], preferred_element_type=f32)
              + jnp.dot(h3, wom_ref[...], preferred_element_type=f32)
              + bo_ref[...])
    out_ref[...] = logits[:, 0]


def _mlp(mu, mi, gu, gi, w1a, w1b, b1, w2, b2, w3, b3, wog, wom, bo):
    n_blocks = _BATCH // _BLK
    emb_spec = pl.BlockSpec((_BLK, _D), lambda i: (i, 0))

    def full(a):
        return pl.BlockSpec(a.shape, lambda i: (0,) * a.ndim)

    return pl.pallas_call(
        _mlp_body,
        grid=(n_blocks,),
        in_specs=[emb_spec, emb_spec, emb_spec, emb_spec,
                  full(w1a), full(w1b), full(b1), full(w2), full(b2),
                  full(w3), full(b3), full(wog), full(wom), full(bo)],
        out_specs=pl.BlockSpec((_BLK,), lambda i: (i,)),
        out_shape=jax.ShapeDtypeStruct((_BATCH,), jnp.float32),
    )(mu, mi, gu, gi, w1a, w1b, b1, w2, b2, w3, b3, wog, wom, bo)


def kernel(x, mlp_user_emb, mlp_item_emb, gmf_user_emb, gmf_item_emb,
           W1, b1, W2, b2, W3, b3, W_out, b_out):
    uid = x[:, 0]
    iid = x[:, 1]
    eye = jnp.eye(_D, dtype=jnp.float32)
    repack = lambda t: _repack_built()(t.T, eye)
    mu, mi, gu, gi = _gather4_built()(
        uid, iid, repack(mlp_user_emb), repack(mlp_item_emb),
        repack(gmf_user_emb), repack(gmf_item_emb))
    return _mlp(mu, mi, gu, gi,
                W1[:_D], W1[_D:], b1.reshape(1, -1),
                W2, b2.reshape(1, -1), W3, b3.reshape(1, -1),
                W_out[:_D], W_out[_D:], b_out.reshape(1, 1))
